# baseline (device time: 13638 ns/iter reference)
import jax
import jax.numpy as jnp
from jax import lax
from jax.experimental import pallas as pl
from jax.experimental.pallas import tpu as pltpu

_CHUNK = 128


def _a2av(x, dest_row):
    m, n = x.shape
    k_max = m // _CHUNK

    def body(dest_ref, x_ref, out_ref, xb_ref, sendbuf_ref, staging_ref,
             send_sems, recv_sems):
        my_x = lax.axis_index("x")
        my_y = lax.axis_index("y")
        peer = (1 - my_x, my_y)

        barrier = pltpu.get_barrier_semaphore()
        pl.semaphore_signal(
            barrier, inc=1, device_id=peer, device_id_type=pl.DeviceIdType.MESH
        )

        dest = dest_ref[:]
        ind0 = (dest == 0).astype(jnp.int32)
        c0v = jnp.sum(ind0)

        col = lax.broadcasted_iota(jnp.int32, (1, m), 1)
        acc = ind0
        shift = 1
        while shift < m:
            acc = acc + jnp.where(col >= shift, pltpu.roll(acc, shift, 1), 0)
            shift *= 2
        rank0 = acc - ind0

        col = lax.broadcasted_iota(jnp.int32, (1, m), 1)
        out_pos = jnp.where(dest == 0, rank0, c0v + (col - rank0))
        pos2 = out_pos - c0v
        pos2 = jnp.where(pos2 < 0, pos2 + m, pos2)

        s = jnp.where(my_x == 0, m - c0v, c0v)
        send_base = jnp.where(my_x == 0, 0, m - c0v)
        recv_base = jnp.where(my_x == 0, c0v, 0)
        keep_lo = jnp.where(my_x == 0, 0, c0v)
        keep_hi = jnp.where(my_x == 0, c0v, m)

        xb_ref[:] = x_ref[:].astype(jnp.bfloat16)

        pl.semaphore_wait(barrier, 1)

        rdmas_out = []
        rdmas_stg = []
        for k in range(k_max):
            rdma_out = pltpu.make_async_remote_copy(
                src_ref=sendbuf_ref.at[pl.ds(k * _CHUNK, _CHUNK)],
                dst_ref=out_ref.at[pl.ds(k * _CHUNK, _CHUNK)],
                send_sem=send_sems.at[k],
                recv_sem=recv_sems.at[k],
                device_id=peer,
                device_id_type=pl.DeviceIdType.MESH,
            )
            rdma_stg = pltpu.make_async_remote_copy(
                src_ref=sendbuf_ref.at[pl.ds(k * _CHUNK, _CHUNK)],
                dst_ref=staging_ref.at[pl.ds(k * _CHUNK, _CHUNK)],
                send_sem=send_sems.at[k],
                recv_sem=recv_sems.at[k],
                device_id=peer,
                device_id_type=pl.DeviceIdType.MESH,
            )
            rdmas_out.append(rdma_out)
            rdmas_stg.append(rdma_stg)

            lo, hi = k * _CHUNK, k * _CHUNK + _CHUNK
            in_window = (hi > send_base) & (lo < send_base + s)
            interior = (lo >= send_base) & (hi <= send_base + s)

            @pl.when(in_window)
            def _(k=k, lo=lo):
                rows = lo + lax.broadcasted_iota(jnp.int32, (_CHUNK, m), 0)
                onehot = (rows == pos2).astype(jnp.bfloat16)
                sendbuf_ref[pl.ds(lo, _CHUNK)] = jnp.dot(
                    onehot, xb_ref[:], preferred_element_type=jnp.float32
                ).astype(jnp.bfloat16)

            @pl.when(in_window & interior)
            def _(rdma=rdma_out):
                rdma.start()

            @pl.when(in_window & ~interior)
            def _(rdma=rdma_stg):
                rdma.start()

        for k in range(k_max):
            lo, hi = k * _CHUNK, k * _CHUNK + _CHUNK
            keep_active = (hi > keep_lo) & (lo < keep_hi)

            @pl.when(keep_active)
            def _(k=k, lo=lo):
                rows = lo + lax.broadcasted_iota(jnp.int32, (_CHUNK, m), 0)
                onehot = (rows == out_pos).astype(jnp.bfloat16)
                out_ref[pl.ds(lo, _CHUNK)] = jnp.dot(
                    onehot, xb_ref[:], preferred_element_type=jnp.float32
                ).astype(jnp.bfloat16)

        rows1 = lax.broadcasted_iota(jnp.int32, (_CHUNK, 1), 0)
        for k in range(k_max):
            lo, hi = k * _CHUNK, k * _CHUNK + _CHUNK
            in_window = (hi > recv_base) & (lo < recv_base + s)
            interior = (lo >= recv_base) & (hi <= recv_base + s)

            @pl.when(in_window)
            def _(rdma=rdmas_stg[k]):
                rdma.wait_recv()

            @pl.when(in_window & ~interior)
            def _(k=k, lo=lo):
                rows = lo + rows1
                keep = (rows >= keep_lo) & (rows < keep_hi)
                out_ref[pl.ds(lo, _CHUNK)] = jnp.where(
                    keep,
                    out_ref[pl.ds(lo, _CHUNK)],
                    staging_ref[pl.ds(lo, _CHUNK)],
                )

        for k in range(k_max):
            lo, hi = k * _CHUNK, k * _CHUNK + _CHUNK
            in_window = (hi > send_base) & (lo < send_base + s)

            @pl.when(in_window)
            def _(rdma=rdmas_stg[k]):
                rdma.wait_send()

    return pl.pallas_call(
        body,
        out_shape=jax.ShapeDtypeStruct((m, n), jnp.bfloat16),
        in_specs=[
            pl.BlockSpec(memory_space=pltpu.VMEM),
            pl.BlockSpec(memory_space=pltpu.VMEM),
        ],
        out_specs=pl.BlockSpec(memory_space=pltpu.VMEM),
        scratch_shapes=[
            pltpu.VMEM((m, n), jnp.bfloat16),
            pltpu.VMEM((m, n), jnp.bfloat16),
            pltpu.VMEM((m, n), jnp.bfloat16),
            pltpu.SemaphoreType.DMA((k_max,)),
            pltpu.SemaphoreType.DMA((k_max,)),
        ],
        compiler_params=pltpu.CompilerParams(collective_id=0),
    )(dest_row, x)


def kernel(x, dest):
    return _a2av(x, dest.reshape(1, -1))


# device time: 13123 ns/iter; 1.0392x vs baseline; 1.0392x over previous
import jax
import jax.numpy as jnp
from jax import lax
from jax.experimental import pallas as pl
from jax.experimental.pallas import tpu as pltpu

_CHUNK = 128


def _a2av(x, dest_row):
    m, n = x.shape
    k_max = m // _CHUNK

    def body(dest_ref, xb_ref, out_ref, sendbuf_ref, staging_ref,
             send_sems, recv_sems):
        my_x = lax.axis_index("x")
        my_y = lax.axis_index("y")
        peer = (1 - my_x, my_y)

        barrier = pltpu.get_barrier_semaphore()
        pl.semaphore_signal(
            barrier, inc=1, device_id=peer, device_id_type=pl.DeviceIdType.MESH
        )

        dest = dest_ref[:]
        ind0 = (dest == 0).astype(jnp.int32)
        c0v = jnp.sum(ind0)

        col = lax.broadcasted_iota(jnp.int32, (1, m), 1)
        acc = ind0
        shift = 1
        while shift < m:
            acc = acc + jnp.where(col >= shift, pltpu.roll(acc, shift, 1), 0)
            shift *= 2
        rank0 = acc - ind0

        col = lax.broadcasted_iota(jnp.int32, (1, m), 1)
        out_pos = jnp.where(dest == 0, rank0, c0v + (col - rank0))
        pos2 = out_pos - c0v
        pos2 = jnp.where(pos2 < 0, pos2 + m, pos2)

        s = jnp.where(my_x == 0, m - c0v, c0v)
        send_base = jnp.where(my_x == 0, 0, m - c0v)
        recv_base = jnp.where(my_x == 0, c0v, 0)
        keep_lo = jnp.where(my_x == 0, 0, c0v)
        keep_hi = jnp.where(my_x == 0, c0v, m)

        pl.semaphore_wait(barrier, 1)

        rdmas_out = []
        rdmas_stg = []
        for k in range(k_max):
            rdma_out = pltpu.make_async_remote_copy(
                src_ref=sendbuf_ref.at[pl.ds(k * _CHUNK, _CHUNK)],
                dst_ref=out_ref.at[pl.ds(k * _CHUNK, _CHUNK)],
                send_sem=send_sems.at[k],
                recv_sem=recv_sems.at[k],
                device_id=peer,
                device_id_type=pl.DeviceIdType.MESH,
            )
            rdma_stg = pltpu.make_async_remote_copy(
                src_ref=sendbuf_ref.at[pl.ds(k * _CHUNK, _CHUNK)],
                dst_ref=staging_ref.at[pl.ds(k * _CHUNK, _CHUNK)],
                send_sem=send_sems.at[k],
                recv_sem=recv_sems.at[k],
                device_id=peer,
                device_id_type=pl.DeviceIdType.MESH,
            )
            rdmas_out.append(rdma_out)
            rdmas_stg.append(rdma_stg)

            lo, hi = k * _CHUNK, k * _CHUNK + _CHUNK
            in_window = (hi > send_base) & (lo < send_base + s)
            interior = (lo >= send_base) & (hi <= send_base + s)

            @pl.when(in_window)
            def _(k=k, lo=lo):
                rows = lo + lax.broadcasted_iota(jnp.int32, (_CHUNK, m), 0)
                onehot = (rows == pos2).astype(jnp.bfloat16)
                sendbuf_ref[pl.ds(lo, _CHUNK)] = jnp.dot(
                    onehot, xb_ref[:], preferred_element_type=jnp.float32
                ).astype(jnp.bfloat16)

            @pl.when(in_window & interior)
            def _(rdma=rdma_out):
                rdma.start()

            @pl.when(in_window & ~interior)
            def _(rdma=rdma_stg):
                rdma.start()

        for k in range(k_max):
            lo, hi = k * _CHUNK, k * _CHUNK + _CHUNK
            keep_active = (hi > keep_lo) & (lo < keep_hi)

            @pl.when(keep_active)
            def _(k=k, lo=lo):
                rows = lo + lax.broadcasted_iota(jnp.int32, (_CHUNK, m), 0)
                onehot = (rows == out_pos).astype(jnp.bfloat16)
                out_ref[pl.ds(lo, _CHUNK)] = jnp.dot(
                    onehot, xb_ref[:], preferred_element_type=jnp.float32
                ).astype(jnp.bfloat16)

        rows1 = lax.broadcasted_iota(jnp.int32, (_CHUNK, 1), 0)
        for k in range(k_max):
            lo, hi = k * _CHUNK, k * _CHUNK + _CHUNK
            in_window = (hi > recv_base) & (lo < recv_base + s)
            interior = (lo >= recv_base) & (hi <= recv_base + s)

            @pl.when(in_window)
            def _(rdma=rdmas_stg[k]):
                rdma.wait_recv()

            @pl.when(in_window & ~interior)
            def _(k=k, lo=lo):
                rows = lo + rows1
                keep = (rows >= keep_lo) & (rows < keep_hi)
                out_ref[pl.ds(lo, _CHUNK)] = jnp.where(
                    keep,
                    out_ref[pl.ds(lo, _CHUNK)],
                    staging_ref[pl.ds(lo, _CHUNK)],
                )

        for k in range(k_max):
            lo, hi = k * _CHUNK, k * _CHUNK + _CHUNK
            in_window = (hi > send_base) & (lo < send_base + s)

            @pl.when(in_window)
            def _(rdma=rdmas_stg[k]):
                rdma.wait_send()

    return pl.pallas_call(
        body,
        out_shape=jax.ShapeDtypeStruct((m, n), jnp.bfloat16),
        in_specs=[
            pl.BlockSpec(memory_space=pltpu.VMEM),
            pl.BlockSpec(memory_space=pltpu.VMEM),
        ],
        out_specs=pl.BlockSpec(memory_space=pltpu.VMEM),
        scratch_shapes=[
            pltpu.VMEM((m, n), jnp.bfloat16),
            pltpu.VMEM((m, n), jnp.bfloat16),
            pltpu.SemaphoreType.DMA((k_max,)),
            pltpu.SemaphoreType.DMA((k_max,)),
        ],
        compiler_params=pltpu.CompilerParams(collective_id=0),
    )(dest_row, x)


def kernel(x, dest):
    return _a2av(x.astype(jnp.bfloat16), dest.reshape(1, -1))
